# Initial kernel scaffold; baseline (speedup 1.0000x reference)
import jax
import jax.numpy as jnp
from jax.experimental import pallas as pl


def kernel(x, Patt, b, c, h, w):
    raise NotImplementedError("write your pallas kernel here")


# speedup vs baseline: 6.7255x; 6.7255x over previous
import functools

import jax
import jax.numpy as jnp
from jax import lax
from jax.experimental import pallas as pl
from jax.experimental.pallas import tpu as pltpu
from jax.experimental.pallas import tpu_sc as plsc

_N0 = 2500.0
_LANES = 16


def _preprocess_sc(xf, patt, *, num_cores, num_subcores, interpret=False):
    num_workers = num_cores * num_subcores
    total_in = xf.shape[0]
    m = patt.shape[0]
    total_out = total_in // 2
    in_per_w = total_in // num_workers
    out_per_w = total_out // num_workers
    vecs_per_w = out_per_w // _LANES
    vecs_per_row = m // _LANES

    mesh = plsc.VectorSubcoreMesh(
        core_axis_name="c", subcore_axis_name="s",
        num_cores=num_cores, num_subcores=num_subcores,
    )

    @functools.partial(
        pl.kernel,
        out_type=jax.ShapeDtypeStruct((total_out,), jnp.float32),
        mesh=mesh,
        scratch_types=[
            pltpu.VMEM((in_per_w,), jnp.float32),
            pltpu.VMEM((m,), jnp.float32),
            pltpu.VMEM((out_per_w,), jnp.float32),
        ],
        compiler_params=pltpu.CompilerParams(needs_layout_passes=False),
        interpret=interpret,
    )
    def run(x_hbm, patt_hbm, out_hbm, x_v, patt_v, out_v):
        wid = lax.axis_index("s") * num_cores + lax.axis_index("c")
        pltpu.sync_copy(x_hbm.at[pl.ds(wid * in_per_w, in_per_w)], x_v)
        pltpu.sync_copy(patt_hbm, patt_v)

        iota = lax.iota(jnp.int32, _LANES)
        scale = jnp.float32(2.0 / _N0)

        def body(v, carry):
            e_idx = v * 32 + 2 * iota
            even = plsc.load_gather(x_v, [e_idx])
            odd = plsc.load_gather(x_v, [e_idx + 1])
            pm = lax.rem(v, vecs_per_row) * _LANES
            p = patt_v[pl.ds(pm, _LANES)]
            out_v[pl.ds(v * _LANES, _LANES)] = (even - odd) * scale - p
            return carry

        lax.fori_loop(0, vecs_per_w, body, 0)
        pltpu.sync_copy(out_v, out_hbm.at[pl.ds(wid * out_per_w, out_per_w)])

    return run(xf, patt)


def kernel(x, Patt, b, c, h, w):
    bs, cs, two_m = x.shape
    m = Patt.shape[0]
    xf = jnp.reshape(x, (bs * cs * two_m,))
    info = plsc.get_sparse_core_info()
    out = _preprocess_sc(xf, Patt.astype(jnp.float32),
                         num_cores=info.num_cores,
                         num_subcores=info.num_subcores)
    return jnp.reshape(out, (bs, cs, m))


# speedup vs baseline: 8.2790x; 1.2310x over previous
import functools

import jax
import jax.numpy as jnp
from jax import lax
from jax.experimental import pallas as pl
from jax.experimental.pallas import tpu as pltpu
from jax.experimental.pallas import tpu_sc as plsc

_N0 = 2500.0
_LANES = 16


def _preprocess_sc(xf, patt, *, num_cores, num_subcores, interpret=False):
    num_workers = num_cores * num_subcores
    total_in = xf.shape[0]
    m = patt.shape[0]
    total_out = total_in // 2
    in_per_w = total_in // num_workers
    out_per_w = total_out // num_workers
    vecs_per_w = out_per_w // _LANES
    vecs_per_row = m // _LANES

    mesh = plsc.VectorSubcoreMesh(
        core_axis_name="c", subcore_axis_name="s",
        num_cores=num_cores, num_subcores=num_subcores,
    )

    @functools.partial(
        pl.kernel,
        out_type=jax.ShapeDtypeStruct((total_out,), jnp.float32),
        mesh=mesh,
        scratch_types=[
            pltpu.VMEM((in_per_w,), jnp.float32),
            pltpu.VMEM((m,), jnp.float32),
            pltpu.VMEM((out_per_w,), jnp.float32),
        ],
        compiler_params=pltpu.CompilerParams(needs_layout_passes=False),
        interpret=interpret,
    )
    def run(x_hbm, patt_hbm, out_hbm, x_v, patt_v, out_v):
        wid = lax.axis_index("s") * num_cores + lax.axis_index("c")
        pltpu.sync_copy(x_hbm.at[pl.ds(wid * in_per_w, in_per_w)], x_v)
        pltpu.sync_copy(patt_hbm, patt_v)

        even_iota = 2 * lax.iota(jnp.int32, _LANES)
        odd_iota = even_iota + 1
        scale = jnp.float32(2.0 / _N0)
        rows_per_w = vecs_per_w // vecs_per_row
        row_in = 2 * m

        for r in range(rows_per_w):
            @plsc.parallel_loop(0, vecs_per_row, 1, unroll=8)
            def row_body(j, r=r):
                base = r * row_in + j * 32
                even = plsc.load_gather(x_v, [base + even_iota])
                odd = plsc.load_gather(x_v, [base + odd_iota])
                p = patt_v[pl.ds(j * _LANES, _LANES)]
                out_v[pl.ds(r * m + j * _LANES, _LANES)] = (
                    (even - odd) * scale - p)
        pltpu.sync_copy(out_v, out_hbm.at[pl.ds(wid * out_per_w, out_per_w)])

    return run(xf, patt)


def kernel(x, Patt, b, c, h, w):
    bs, cs, two_m = x.shape
    m = Patt.shape[0]
    xf = jnp.reshape(x, (bs * cs * two_m,))
    info = plsc.get_sparse_core_info()
    out = _preprocess_sc(xf, Patt.astype(jnp.float32),
                         num_cores=info.num_cores,
                         num_subcores=info.num_subcores)
    return jnp.reshape(out, (bs, cs, m))


# speedup vs baseline: 8.4151x; 1.0164x over previous
import functools

import jax
import jax.numpy as jnp
from jax import lax
from jax.experimental import pallas as pl
from jax.experimental.pallas import tpu as pltpu
from jax.experimental.pallas import tpu_sc as plsc

_N0 = 2500.0
_LANES = 16


def _preprocess_sc(xf, patt, *, num_cores, num_subcores, interpret=False):
    num_workers = num_cores * num_subcores
    total_in = xf.shape[0]
    m = patt.shape[0]
    total_out = total_in // 2
    in_per_w = total_in // num_workers
    out_per_w = total_out // num_workers
    vecs_per_row = m // _LANES
    rows_per_w = out_per_w // m
    row_in = 2 * m

    mesh = plsc.VectorSubcoreMesh(
        core_axis_name="c", subcore_axis_name="s",
        num_cores=num_cores, num_subcores=num_subcores,
    )

    @functools.partial(
        pl.kernel,
        out_type=jax.ShapeDtypeStruct((total_out,), jnp.float32),
        mesh=mesh,
        scratch_types=[
            pltpu.VMEM((row_in,), jnp.float32),
            pltpu.VMEM((row_in,), jnp.float32),
            pltpu.VMEM((m,), jnp.float32),
            pltpu.VMEM((m,), jnp.float32),
            pltpu.VMEM((m,), jnp.float32),
            pltpu.SemaphoreType.DMA,
            pltpu.SemaphoreType.DMA,
            pltpu.SemaphoreType.DMA,
            pltpu.SemaphoreType.DMA,
            pltpu.SemaphoreType.DMA,
        ],
        compiler_params=pltpu.CompilerParams(needs_layout_passes=False),
        interpret=interpret,
    )
    def run(x_hbm, patt_hbm, out_hbm, in0, in1, out0, out1, patt_v,
            si0, si1, so0, so1, sp):
        wid = lax.axis_index("s") * num_cores + lax.axis_index("c")
        row0 = wid * rows_per_w
        ins, outs = [in0, in1], [out0, out1]
        sis, sos = [si0, si1], [so0, so1]

        even_iota = 2 * lax.iota(jnp.int32, _LANES)
        odd_iota = even_iota + 1
        scale = jnp.float32(2.0 / _N0)

        patt_cp = pltpu.async_copy(patt_hbm, patt_v, sp)
        in_cps = [None, None]
        in_cps[0] = pltpu.async_copy(
            x_hbm.at[pl.ds(row0 * row_in, row_in)], in0, si0)
        out_cps = [None, None]

        for r in range(rows_per_w):
            if r + 1 < rows_per_w:
                nb = (r + 1) % 2
                in_cps[nb] = pltpu.async_copy(
                    x_hbm.at[pl.ds((row0 + r + 1) * row_in, row_in)],
                    ins[nb], sis[nb])
            b = r % 2
            in_cps[b].wait()
            if r == 0:
                patt_cp.wait()
            if r >= 2:
                out_cps[b].wait()
            xb, ob = ins[b], outs[b]

            @plsc.parallel_loop(0, vecs_per_row, 1, unroll=8)
            def row_body(j, xb=xb, ob=ob):
                base = j * 32
                even = plsc.load_gather(xb, [base + even_iota])
                odd = plsc.load_gather(xb, [base + odd_iota])
                p = patt_v[pl.ds(j * _LANES, _LANES)]
                ob[pl.ds(j * _LANES, _LANES)] = (even - odd) * scale - p

            out_cps[b] = pltpu.async_copy(
                ob, out_hbm.at[pl.ds((row0 + r) * m, m)], sos[b])

        for cp in out_cps:
            if cp is not None:
                cp.wait()

    return run(xf, patt)


def kernel(x, Patt, b, c, h, w):
    bs, cs, two_m = x.shape
    m = Patt.shape[0]
    xf = jnp.reshape(x, (bs * cs * two_m,))
    info = plsc.get_sparse_core_info()
    out = _preprocess_sc(xf, Patt.astype(jnp.float32),
                         num_cores=info.num_cores,
                         num_subcores=info.num_subcores)
    return jnp.reshape(out, (bs, cs, m))


# speedup vs baseline: 8.4253x; 1.0012x over previous
import functools

import jax
import jax.numpy as jnp
from jax import lax
from jax.experimental import pallas as pl
from jax.experimental.pallas import tpu as pltpu
from jax.experimental.pallas import tpu_sc as plsc

_N0 = 2500.0
_LANES = 16


def _preprocess_sc(xf, patt, *, num_cores, num_subcores, interpret=False):
    num_workers = num_cores * num_subcores
    total_in = xf.shape[0]
    m = patt.shape[0]
    total_out = total_in // 2
    in_per_w = total_in // num_workers
    out_per_w = total_out // num_workers
    vecs_per_row = m // _LANES
    rows_per_w = out_per_w // m
    row_in = 2 * m

    mesh = plsc.VectorSubcoreMesh(
        core_axis_name="c", subcore_axis_name="s",
        num_cores=num_cores, num_subcores=num_subcores,
    )

    @functools.partial(
        pl.kernel,
        out_type=jax.ShapeDtypeStruct((total_out,), jnp.float32),
        mesh=mesh,
        scratch_types=[
            pltpu.VMEM((row_in,), jnp.float32),
            pltpu.VMEM((row_in,), jnp.float32),
            pltpu.VMEM((m,), jnp.float32),
            pltpu.VMEM((m,), jnp.float32),
            pltpu.VMEM((m,), jnp.float32),
            pltpu.SemaphoreType.DMA,
            pltpu.SemaphoreType.DMA,
            pltpu.SemaphoreType.DMA,
            pltpu.SemaphoreType.DMA,
            pltpu.SemaphoreType.DMA,
        ],
        compiler_params=pltpu.CompilerParams(
            needs_layout_passes=False,
            disable_semaphore_checks=True,
            skip_device_barrier=True,
        ),
        interpret=interpret,
    )
    def run(x_hbm, patt_hbm, out_hbm, in0, in1, out0, out1, patt_v,
            si0, si1, so0, so1, sp):
        wid = lax.axis_index("s") * num_cores + lax.axis_index("c")
        row0 = wid * rows_per_w
        ins, outs = [in0, in1], [out0, out1]
        sis, sos = [si0, si1], [so0, so1]

        even_iota = 2 * lax.iota(jnp.int32, _LANES)
        odd_iota = even_iota + 1
        scale = jnp.float32(2.0 / _N0)

        patt_cp = pltpu.async_copy(patt_hbm, patt_v, sp)
        in_cps = [None, None]
        in_cps[0] = pltpu.async_copy(
            x_hbm.at[pl.ds(row0 * row_in, row_in)], in0, si0)
        out_cps = [None, None]

        for r in range(rows_per_w):
            if r + 1 < rows_per_w:
                nb = (r + 1) % 2
                in_cps[nb] = pltpu.async_copy(
                    x_hbm.at[pl.ds((row0 + r + 1) * row_in, row_in)],
                    ins[nb], sis[nb])
            b = r % 2
            in_cps[b].wait()
            if r == 0:
                patt_cp.wait()
            if r >= 2:
                out_cps[b].wait()
            xb, ob = ins[b], outs[b]

            @plsc.parallel_loop(0, vecs_per_row, 1, unroll=8)
            def row_body(j, xb=xb, ob=ob):
                base = j * 32
                even = plsc.load_gather(xb, [base + even_iota])
                odd = plsc.load_gather(xb, [base + odd_iota])
                p = patt_v[pl.ds(j * _LANES, _LANES)]
                ob[pl.ds(j * _LANES, _LANES)] = (even - odd) * scale - p

            out_cps[b] = pltpu.async_copy(
                ob, out_hbm.at[pl.ds((row0 + r) * m, m)], sos[b])

        for cp in out_cps:
            if cp is not None:
                cp.wait()

    return run(xf, patt)


def kernel(x, Patt, b, c, h, w):
    bs, cs, two_m = x.shape
    m = Patt.shape[0]
    xf = jnp.reshape(x, (bs * cs * two_m,))
    info = plsc.get_sparse_core_info()
    out = _preprocess_sc(xf, Patt.astype(jnp.float32),
                         num_cores=info.num_cores,
                         num_subcores=info.num_subcores)
    return jnp.reshape(out, (bs, cs, m))
